# baseline (speedup 1.0000x reference)
import functools
import jax
import jax.numpy as jnp
from jax import lax
from jax.experimental import pallas as pl
from jax.experimental.pallas import tpu as pltpu
from jax.experimental.pallas import tpu_sc as plsc

EMB = 64
SCALE = 8.0
NC, NS = 2, 16
NW = NC * NS
CHUNK = 128
L = 16


def _make_kernel(B):
    assert B % (NW * CHUNK) == 0
    bpw = B // NW
    nch = bpw // CHUNK
    mesh = plsc.VectorSubcoreMesh(core_axis_name="c", subcore_axis_name="s")

    @functools.partial(
        pl.kernel,
        out_type=jax.ShapeDtypeStruct((B, EMB), jnp.float32),
        mesh=mesh,
        compiler_params=pltpu.CompilerParams(use_tc_tiling_on_sc=False),
        scratch_types=[
            pltpu.VMEM((nch, CHUNK), jnp.int32),
            pltpu.VMEM((CHUNK, EMB), jnp.float32),
            pltpu.SemaphoreType.DMA,
        ],
    )
    def k(idx_hbm, table_hbm, out_hbm, idx_v, gbuf, sem):
        wid = lax.axis_index("s") * NC + lax.axis_index("c")
        base = wid * bpw
        pltpu.sync_copy(idx_hbm.at[wid], idx_v)

        @pl.loop(0, nch)
        def chunk_loop(j):
            pltpu.async_copy(table_hbm.at[idx_v.at[j]], gbuf, sem).wait()

            @pl.loop(0, CHUNK)
            def row_loop(r):
                for c in range(0, EMB, L):
                    gbuf[r, pl.ds(c, L)] = gbuf[r, pl.ds(c, L)] * SCALE

            pltpu.sync_copy(gbuf, out_hbm.at[pl.ds(base + j * CHUNK, CHUNK)])

    return k


def kernel(tokens, table):
    S0, S1 = tokens.shape
    B = S0 * S1
    idx = tokens.reshape(NW, B // (NW * CHUNK), CHUNK).astype(jnp.int32)
    out = _make_kernel(B)(idx, table)
    return out.reshape(S0, S1, EMB)


# speedup vs baseline: 1.2065x; 1.2065x over previous
import functools
import jax
import jax.numpy as jnp
from jax import lax
from jax.experimental import pallas as pl
from jax.experimental.pallas import tpu as pltpu
from jax.experimental.pallas import tpu_sc as plsc

EMB = 64
SCALE = 8.0
NC, NS = 2, 16
NW = NC * NS
CHUNK = 128
L = 16
NBUF = 4


def _make_kernel(B):
    assert B % (NW * CHUNK) == 0
    bpw = B // NW
    nch = bpw // CHUNK
    assert nch % NBUF == 0
    mesh = plsc.VectorSubcoreMesh(core_axis_name="c", subcore_axis_name="s")

    @functools.partial(
        pl.kernel,
        out_type=jax.ShapeDtypeStruct((B, EMB), jnp.float32),
        mesh=mesh,
        compiler_params=pltpu.CompilerParams(use_tc_tiling_on_sc=False),
        scratch_types=[
            pltpu.VMEM((nch, CHUNK), jnp.int32),
            pltpu.VMEM((NBUF, CHUNK, EMB), jnp.float32),
            pltpu.VMEM((NBUF, CHUNK, EMB), jnp.float32),
            pltpu.SemaphoreType.DMA((NBUF,)),
            pltpu.SemaphoreType.DMA((NBUF,)),
        ],
    )
    def k(idx_hbm, table_hbm, out_hbm, idx_v, gbufs, wbufs, gsem, wsem):
        wid = lax.axis_index("s") * NC + lax.axis_index("c")
        base = wid * bpw
        pltpu.sync_copy(idx_hbm.at[wid], idx_v)

        for b in range(NBUF):
            pltpu.async_copy(table_hbm.at[idx_v.at[b]], gbufs.at[b], gsem.at[b])

        @pl.loop(0, nch, step=NBUF)
        def outer(j):
            for b in range(NBUF):
                jj = j + b
                pltpu.make_async_copy(
                    table_hbm.at[idx_v.at[0]], gbufs.at[b], gsem.at[b]
                ).wait()
                @pl.when(j > 0)
                def _():
                    pltpu.make_async_copy(
                        wbufs.at[b], out_hbm.at[pl.ds(0, CHUNK)], wsem.at[b]
                    ).wait()

                @plsc.parallel_loop(0, CHUNK, unroll=4)
                def rows(r):
                    for c in range(0, EMB, L):
                        wbufs[b, r, pl.ds(c, L)] = gbufs[b, r, pl.ds(c, L)] * SCALE

                pltpu.async_copy(
                    wbufs.at[b],
                    out_hbm.at[pl.ds(base + jj * CHUNK, CHUNK)],
                    wsem.at[b],
                )

                @pl.when(jj + NBUF < nch)
                def _():
                    pltpu.async_copy(
                        table_hbm.at[idx_v.at[jj + NBUF]], gbufs.at[b], gsem.at[b]
                    )

        for b in range(NBUF):
            pltpu.make_async_copy(
                wbufs.at[b], out_hbm.at[pl.ds(0, CHUNK)], wsem.at[b]
            ).wait()

    return k


def kernel(tokens, table):
    S0, S1 = tokens.shape
    B = S0 * S1
    idx = tokens.reshape(NW, B // (NW * CHUNK), CHUNK).astype(jnp.int32)
    out = _make_kernel(B)(idx, table)
    return out.reshape(S0, S1, EMB)
